# baseline (speedup 1.0000x reference)
import functools

import jax
import jax.numpy as jnp
from jax import lax
from jax.experimental import pallas as pl
from jax.experimental.pallas import tpu as pltpu
from jax.experimental.pallas import tpu_sc as plsc

_VOCAB = 50000
_D = 512
_B = 4096
_L = 50

_NC = 2
_NS = 16
_NW = _NC * _NS
_BPW = _B // _NW
_SEGS_PER_CHUNK = 2
_ROWS = _SEGS_PER_CHUNK * _L
_IDS_PAD = 104
_CPW = _BPW // _SEGS_PER_CHUNK
_NITER = _CPW // 2
_IPW = _BPW * _L


def _sc_pool(ids_flat, table):
    mesh = plsc.VectorSubcoreMesh(
        core_axis_name="c", subcore_axis_name="s",
        num_cores=_NC, num_subcores=_NS)

    @functools.partial(
        pl.kernel,
        mesh=mesh,
        out_type=jax.ShapeDtypeStruct((_B, _D), jnp.float32),
        scratch_types=[
            pltpu.VMEM((_IPW + 16,), jnp.int32),
            pltpu.VMEM((_CPW * _IDS_PAD + 8,), jnp.int32),
            pltpu.VMEM((_IDS_PAD, _D), jnp.float32),
            pltpu.VMEM((_IDS_PAD, _D), jnp.float32),
            pltpu.VMEM((2 * _SEGS_PER_CHUNK, _D), jnp.float32),
            pltpu.SemaphoreType.DMA,
            pltpu.SemaphoreType.DMA,
            pltpu.SemaphoreType.DMA,
            pltpu.SemaphoreType.DMA,
        ],
    )
    def k(ids_hbm, table_hbm, out_hbm, ids_raw, idx_v, rows0, rows1, acc_v,
          sem0, sem1, osem0, osem1):
        wid = lax.axis_index("s") * _NC + lax.axis_index("c")

        pltpu.sync_copy(ids_hbm.at[pl.ds(wid * _IPW, _IPW)],
                        ids_raw.at[pl.ds(0, _IPW)])
        zero16i = jnp.zeros((16,), jnp.int32)
        ids_raw[pl.ds(_IPW, 16)] = zero16i

        def repack(c, carry):
            for j in range(7):
                idx_v[pl.ds(c * _IDS_PAD + 16 * j, 16)] = (
                    ids_raw[pl.ds(c * _ROWS + 16 * j, 16)])
            return carry

        lax.fori_loop(0, _CPW, repack, 0)

        def gather_start(c, buf, sem):
            pltpu.async_copy(
                table_hbm.at[idx_v.at[pl.ds(c * _IDS_PAD, _IDS_PAD)]], buf, sem)

        def gather_wait(c, buf, sem):
            pltpu.make_async_copy(
                table_hbm.at[idx_v.at[pl.ds(c * _IDS_PAD, _IDS_PAD)]], buf,
                sem).wait()

        zero16 = jnp.zeros((16,), jnp.float32)

        def reduce_chunk(buf, off):
            for s in range(_SEGS_PER_CHUNK):
                for j in range(_D // 16):
                    acc_v[off + s, pl.ds(16 * j, 16)] = zero16

            @plsc.parallel_loop(0, (_L - 2) // 4)
            def body_l(l):
                for s in range(_SEGS_PER_CHUNK):
                    r0 = s * _L + 4 * l
                    for j in range(_D // 16):
                        sl = pl.ds(16 * j, 16)
                        t = ((buf[r0, sl] + buf[r0 + 1, sl]) +
                             (buf[r0 + 2, sl] + buf[r0 + 3, sl]))
                        plsc.addupdate(acc_v.at[off + s, sl], t)

            for s in range(_SEGS_PER_CHUNK):
                r0 = s * _L + (_L - 2)
                for j in range(_D // 16):
                    sl = pl.ds(16 * j, 16)
                    plsc.addupdate(acc_v.at[off + s, sl],
                                   buf[r0, sl] + buf[r0 + 1, sl])

        gather_start(0, rows0, sem0)
        base = wid * _BPW
        sc = _SEGS_PER_CHUNK

        def out_start(i, half, osem):
            pltpu.async_copy(
                acc_v.at[pl.ds(half * sc, sc)],
                out_hbm.at[pl.ds(base + i * (2 * sc) + half * sc, sc)], osem)

        def out_wait(half, osem):
            pltpu.make_async_copy(
                acc_v.at[pl.ds(half * sc, sc)],
                out_hbm.at[pl.ds(base, sc)], osem).wait()

        def body(i, carry):
            c0 = 2 * i
            gather_start(c0 + 1, rows1, sem1)
            gather_wait(c0, rows0, sem0)

            @pl.when(i > 0)
            def _():
                out_wait(0, osem0)

            reduce_chunk(rows0, 0)

            @pl.when(i < _NITER - 1)
            def _():
                gather_start(c0 + 2, rows0, sem0)

            out_start(i, 0, osem0)
            gather_wait(c0 + 1, rows1, sem1)

            @pl.when(i > 0)
            def _():
                out_wait(1, osem1)

            reduce_chunk(rows1, sc)
            out_start(i, 1, osem1)
            return carry

        lax.fori_loop(0, _NITER, body, 0)
        out_wait(0, osem0)
        out_wait(1, osem1)

    return k(ids_flat, table)


_NT = (((1,), (1,)), ((), ()))


def _dense_body(sums_ref, mask_ref, w1_ref, b1_ref, g_ref, be_ref,
                w2_ref, b2_ref, wl_ref, bl_ref, wb_ref, bb_ref,
                wr_ref, br_ref, wp_ref, bp_ref,
                sv_ref, lat_ref, bw_ref, rel_ref, pri_ref):
    msum = jnp.sum(mask_ref[...], axis=1, keepdims=True)
    pooled = sums_ref[...] / msum
    h = lax.dot_general(pooled, w1_ref[...], _NT,
                        preferred_element_type=jnp.float32,
                        precision=jax.lax.Precision.HIGHEST) + b1_ref[...]
    mu = jnp.mean(h, axis=-1, keepdims=True)
    hc = h - mu
    var = jnp.mean(hc * hc, axis=-1, keepdims=True)
    hn = hc * jax.lax.rsqrt(var + 1e-5) * g_ref[...] + be_ref[...]
    hg = 0.5 * hn * (1.0 + jax.lax.erf(hn * (2.0 ** -0.5)))
    sv = lax.dot_general(hg, w2_ref[...], _NT,
                         preferred_element_type=jnp.float32,
                         precision=jax.lax.Precision.HIGHEST) + b2_ref[...]
    sv_ref[...] = sv
    for w_ref, b_ref, o_ref in ((wl_ref, bl_ref, lat_ref),
                                (wb_ref, bb_ref, bw_ref),
                                (wr_ref, br_ref, rel_ref),
                                (wp_ref, bp_ref, pri_ref)):
        o_ref[...] = lax.dot_general(
            sv, w_ref[...], _NT,
            preferred_element_type=jnp.float32,
            precision=jax.lax.Precision.HIGHEST) + b_ref[...]


def _dense(sums, mask, W1, b1r, gr, ber, W2, b2r,
           Wl, blr, Wb, bbr, Wr, brr, Wp, bpr):
    bm = 256
    grid = (_B // bm,)
    full = lambda shape: pl.BlockSpec(shape, lambda i: (0,) * len(shape))
    return pl.pallas_call(
        _dense_body,
        grid=grid,
        in_specs=[
            pl.BlockSpec((bm, _D), lambda i: (i, 0)),
            pl.BlockSpec((bm, _L), lambda i: (i, 0)),
            full((_D, _D)),
            full((1, _D)),
            full((1, _D)),
            full((1, _D)),
            full((128, _D)),
            full((1, 128)),
            full((3, 128)),
            full((1, 3)),
            full((3, 128)),
            full((1, 3)),
            full((3, 128)),
            full((1, 3)),
            full((4, 128)),
            full((1, 4)),
        ],
        out_specs=[
            pl.BlockSpec((bm, 128), lambda i: (i, 0)),
            pl.BlockSpec((bm, 3), lambda i: (i, 0)),
            pl.BlockSpec((bm, 3), lambda i: (i, 0)),
            pl.BlockSpec((bm, 3), lambda i: (i, 0)),
            pl.BlockSpec((bm, 4), lambda i: (i, 0)),
        ],
        out_shape=[
            jax.ShapeDtypeStruct((_B, 128), jnp.float32),
            jax.ShapeDtypeStruct((_B, 3), jnp.float32),
            jax.ShapeDtypeStruct((_B, 3), jnp.float32),
            jax.ShapeDtypeStruct((_B, 3), jnp.float32),
            jax.ShapeDtypeStruct((_B, 4), jnp.float32),
        ],
    )(sums, mask, W1, b1r, gr, ber, W2, b2r,
      Wl, blr, Wb, bbr, Wr, brr, Wp, bpr)


def kernel(input_ids, attention_mask, emb_table, W1, b1, gamma, beta,
           W2, b2, Wl, bl, Wb, bb, Wr, br, Wp, bp):
    ids_flat = input_ids.astype(jnp.int32).reshape(-1)

    sums = _sc_pool(ids_flat, emb_table)

    sv, lat, bw, rel, pri = _dense(
        sums, attention_mask, W1,
        b1.reshape(1, _D), gamma.reshape(1, _D), beta.reshape(1, _D),
        W2, b2.reshape(1, 128),
        Wl, bl.reshape(1, 3), Wb, bb.reshape(1, 3),
        Wr, br.reshape(1, 3), Wp, bp.reshape(1, 4))
    return (sv, lat, bw, rel, pri)


# speedup vs baseline: 1.0223x; 1.0223x over previous
import functools

import jax
import jax.numpy as jnp
from jax import lax
from jax.experimental import pallas as pl
from jax.experimental.pallas import tpu as pltpu
from jax.experimental.pallas import tpu_sc as plsc

_VOCAB = 50000
_D = 512
_B = 4096
_L = 50

_NC = 2
_NS = 16
_NW = _NC * _NS
_BPW = _B // _NW
_SEGS_PER_CHUNK = 2
_ROWS = _SEGS_PER_CHUNK * _L
_IDS_PAD = 104
_CPW = _BPW // _SEGS_PER_CHUNK
_NITER = _CPW // 2
_IPW = _BPW * _L


def _sc_pool(ids_flat, table):
    mesh = plsc.VectorSubcoreMesh(
        core_axis_name="c", subcore_axis_name="s",
        num_cores=_NC, num_subcores=_NS)

    @functools.partial(
        pl.kernel,
        mesh=mesh,
        out_type=jax.ShapeDtypeStruct((_B, _D), jnp.float32),
        scratch_types=[
            pltpu.VMEM((_IPW + 16,), jnp.int32),
            pltpu.VMEM((_CPW * _IDS_PAD + 8,), jnp.int32),
            pltpu.VMEM((_IDS_PAD, _D), jnp.float32),
            pltpu.VMEM((_IDS_PAD, _D), jnp.float32),
            pltpu.VMEM((2 * _SEGS_PER_CHUNK, _D), jnp.float32),
            pltpu.SemaphoreType.DMA,
            pltpu.SemaphoreType.DMA,
        ],
    )
    def k(ids_hbm, table_hbm, out_hbm, ids_raw, idx_v, rows0, rows1, acc_v,
          sem0, sem1):
        wid = lax.axis_index("s") * _NC + lax.axis_index("c")

        pltpu.sync_copy(ids_hbm.at[pl.ds(wid * _IPW, _IPW)],
                        ids_raw.at[pl.ds(0, _IPW)])
        zero16i = jnp.zeros((16,), jnp.int32)
        ids_raw[pl.ds(_IPW, 16)] = zero16i

        def repack(c, carry):
            for j in range(7):
                idx_v[pl.ds(c * _IDS_PAD + 16 * j, 16)] = (
                    ids_raw[pl.ds(c * _ROWS + 16 * j, 16)])
            return carry

        lax.fori_loop(0, _CPW, repack, 0)

        def gather_start(c, buf, sem):
            pltpu.async_copy(
                table_hbm.at[idx_v.at[pl.ds(c * _IDS_PAD, _IDS_PAD)]], buf, sem)

        def gather_wait(c, buf, sem):
            pltpu.make_async_copy(
                table_hbm.at[idx_v.at[pl.ds(c * _IDS_PAD, _IDS_PAD)]], buf,
                sem).wait()

        zero16 = jnp.zeros((16,), jnp.float32)

        def reduce_chunk(buf, off):
            for s in range(_SEGS_PER_CHUNK):
                for j in range(_D // 16):
                    acc_v[off + s, pl.ds(16 * j, 16)] = zero16

            @plsc.parallel_loop(0, (_L - 2) // 4)
            def body_l(l):
                for s in range(_SEGS_PER_CHUNK):
                    r0 = s * _L + 4 * l
                    for j in range(_D // 16):
                        sl = pl.ds(16 * j, 16)
                        t = ((buf[r0, sl] + buf[r0 + 1, sl]) +
                             (buf[r0 + 2, sl] + buf[r0 + 3, sl]))
                        plsc.addupdate(acc_v.at[off + s, sl], t)

            for s in range(_SEGS_PER_CHUNK):
                r0 = s * _L + (_L - 2)
                for j in range(_D // 16):
                    sl = pl.ds(16 * j, 16)
                    plsc.addupdate(acc_v.at[off + s, sl],
                                   buf[r0, sl] + buf[r0 + 1, sl])

        gather_start(0, rows0, sem0)

        def body(i, carry):
            c0 = 2 * i
            gather_start(c0 + 1, rows1, sem1)
            gather_wait(c0, rows0, sem0)
            reduce_chunk(rows0, 0)

            @pl.when(i < _NITER - 1)
            def _():
                gather_start(c0 + 2, rows0, sem0)

            gather_wait(c0 + 1, rows1, sem1)
            reduce_chunk(rows1, _SEGS_PER_CHUNK)
            pltpu.sync_copy(
                acc_v,
                out_hbm.at[pl.ds(wid * _BPW + i * (2 * _SEGS_PER_CHUNK),
                                 2 * _SEGS_PER_CHUNK)])
            return carry

        lax.fori_loop(0, _NITER, body, 0)

    return k(ids_flat, table)


_NT = (((1,), (1,)), ((), ()))


def _dense_body(sums_ref, mask_ref, w1_ref, b1_ref, g_ref, be_ref,
                w2_ref, b2_ref, wl_ref, bl_ref, wb_ref, bb_ref,
                wr_ref, br_ref, wp_ref, bp_ref,
                sv_ref, lat_ref, bw_ref, rel_ref, pri_ref):
    msum = jnp.sum(mask_ref[...], axis=1, keepdims=True)
    pooled = sums_ref[...] / msum
    h = lax.dot_general(pooled, w1_ref[...], _NT,
                        preferred_element_type=jnp.float32,
                        precision=jax.lax.Precision.HIGHEST) + b1_ref[...]
    mu = jnp.mean(h, axis=-1, keepdims=True)
    hc = h - mu
    var = jnp.mean(hc * hc, axis=-1, keepdims=True)
    hn = hc * jax.lax.rsqrt(var + 1e-5) * g_ref[...] + be_ref[...]
    hg = 0.5 * hn * (1.0 + jax.lax.erf(hn * (2.0 ** -0.5)))
    sv = lax.dot_general(hg, w2_ref[...], _NT,
                         preferred_element_type=jnp.float32,
                         precision=jax.lax.Precision.HIGHEST) + b2_ref[...]
    sv_ref[...] = sv
    for w_ref, b_ref, o_ref in ((wl_ref, bl_ref, lat_ref),
                                (wb_ref, bb_ref, bw_ref),
                                (wr_ref, br_ref, rel_ref),
                                (wp_ref, bp_ref, pri_ref)):
        o_ref[...] = lax.dot_general(
            sv, w_ref[...], _NT,
            preferred_element_type=jnp.float32,
            precision=jax.lax.Precision.HIGHEST) + b_ref[...]


def _dense(sums, mask, W1, b1r, gr, ber, W2, b2r,
           Wl, blr, Wb, bbr, Wr, brr, Wp, bpr):
    bm = 256
    grid = (_B // bm,)
    full = lambda shape: pl.BlockSpec(shape, lambda i: (0,) * len(shape))
    return pl.pallas_call(
        _dense_body,
        grid=grid,
        in_specs=[
            pl.BlockSpec((bm, _D), lambda i: (i, 0)),
            pl.BlockSpec((bm, _L), lambda i: (i, 0)),
            full((_D, _D)),
            full((1, _D)),
            full((1, _D)),
            full((1, _D)),
            full((128, _D)),
            full((1, 128)),
            full((3, 128)),
            full((1, 3)),
            full((3, 128)),
            full((1, 3)),
            full((3, 128)),
            full((1, 3)),
            full((4, 128)),
            full((1, 4)),
        ],
        out_specs=[
            pl.BlockSpec((bm, 128), lambda i: (i, 0)),
            pl.BlockSpec((bm, 3), lambda i: (i, 0)),
            pl.BlockSpec((bm, 3), lambda i: (i, 0)),
            pl.BlockSpec((bm, 3), lambda i: (i, 0)),
            pl.BlockSpec((bm, 4), lambda i: (i, 0)),
        ],
        out_shape=[
            jax.ShapeDtypeStruct((_B, 128), jnp.float32),
            jax.ShapeDtypeStruct((_B, 3), jnp.float32),
            jax.ShapeDtypeStruct((_B, 3), jnp.float32),
            jax.ShapeDtypeStruct((_B, 3), jnp.float32),
            jax.ShapeDtypeStruct((_B, 4), jnp.float32),
        ],
    )(sums, mask, W1, b1r, gr, ber, W2, b2r,
      Wl, blr, Wb, bbr, Wr, brr, Wp, bpr)


def kernel(input_ids, attention_mask, emb_table, W1, b1, gamma, beta,
           W2, b2, Wl, bl, Wb, bb, Wr, br, Wp, bp):
    ids_flat = input_ids.astype(jnp.int32).reshape(-1)

    sums = _sc_pool(ids_flat, emb_table)

    sv, lat, bw, rel, pri = _dense(
        sums, attention_mask, W1,
        b1.reshape(1, _D), gamma.reshape(1, _D), beta.reshape(1, _D),
        W2, b2.reshape(1, 128),
        Wl, bl.reshape(1, 3), Wb, bb.reshape(1, 3),
        Wr, br.reshape(1, 3), Wp, bp.reshape(1, 4))
    return (sv, lat, bw, rel, pri)
